# Initial kernel scaffold; baseline (speedup 1.0000x reference)
import jax
import jax.numpy as jnp
from jax.experimental import pallas as pl


def kernel(xyz, h, x, knn_indices, Wz_pos, bz_pos, Wz_lin, bz_lin, Wr_pos, br_pos, Wr_lin, br_lin, Wq_pos, bq_pos, Wq_lin, bq_lin):
    raise NotImplementedError("write your pallas kernel here")


# speedup vs baseline: 7.2574x; 7.2574x over previous
import functools

import jax
import jax.numpy as jnp
from jax import lax
from jax.experimental import pallas as pl
from jax.experimental.pallas import tpu as pltpu
from jax.experimental.pallas import tpu_sc as plsc

NW = 32
CHUNK = 256
BN = 512


def _sc_gather(table, idx, ncol, aux=None, aux_ncol=0):
    total = idx.shape[0]
    per_w = total // NW
    iters = per_w // CHUNK
    nc = 2

    out_type = [jax.ShapeDtypeStruct((total, ncol), jnp.float32)]
    scratch = [
        pltpu.VMEM((CHUNK,), jnp.int32),
        pltpu.VMEM((CHUNK, ncol), jnp.float32),
        pltpu.SemaphoreType.DMA,
    ]
    has_aux = aux is not None
    if has_aux:
        out_type.append(jax.ShapeDtypeStruct((total, aux_ncol), jnp.float32))
        scratch.append(pltpu.VMEM((CHUNK, aux_ncol), jnp.float32))
        scratch.append(pltpu.SemaphoreType.DMA)

    mesh = plsc.VectorSubcoreMesh(core_axis_name="c", subcore_axis_name="s")

    @functools.partial(
        pl.kernel, mesh=mesh, out_type=out_type, scratch_types=scratch,
    )
    def gather_k(idx_hbm, tab_hbm, *rest):
        if has_aux:
            aux_hbm, out_hbm, aux_out, idx_v, rows_v, sem, aux_v, sem2 = rest
        else:
            out_hbm, idx_v, rows_v, sem = rest
        wid = lax.axis_index("s") * nc + lax.axis_index("c")
        base = wid * per_w

        def body(i, carry):
            start = base + i * CHUNK
            pltpu.sync_copy(idx_hbm.at[pl.ds(start, CHUNK)], idx_v)
            cp = pltpu.async_copy(tab_hbm.at[idx_v], rows_v, sem)
            if has_aux:
                cp2 = pltpu.async_copy(aux_hbm.at[idx_v], aux_v, sem2)
            cp.wait()
            pltpu.sync_copy(rows_v, out_hbm.at[pl.ds(start, CHUNK)])
            if has_aux:
                cp2.wait()
                pltpu.sync_copy(aux_v, aux_out.at[pl.ds(start, CHUNK)])
            return carry

        lax.fori_loop(0, iters, body, 0)

    if has_aux:
        return gather_k(idx, table, aux)
    return gather_k(idx, table)


def kernel(xyz, h, x, knn_indices,
           Wz_pos, bz_pos, Wz_lin, bz_lin,
           Wr_pos, br_pos, Wr_lin, br_lin,
           Wq_pos, bq_pos, Wq_lin, bq_lin):
    h = h.astype(jnp.float32)
    x = x.astype(jnp.float32)
    B, N, _ = xyz.shape
    k = knn_indices.shape[2]
    H = h.shape[1]
    IN = x.shape[1]
    CIN = H + IN

    unit = 2048
    Npad = -(-N // unit) * unit
    NKpad = Npad * k

    hT = h[0].T
    hxT = jnp.concatenate([hT, x[0].T], axis=1)
    hxT = jnp.pad(hxT, ((0, Npad - N), (0, 0)))
    hTp = hxT[:, :H]
    xyzp = jnp.pad(xyz[0], ((0, Npad - N), (0, 125)))
    idx_flat = jnp.pad(knn_indices[0].astype(jnp.int32).reshape(N * k),
                       (0, NKpad - N * k))

    nbr_hx, nbr_xyz = _sc_gather(hxT, idx_flat, CIN, aux=xyzp, aux_ncol=128)

    def padpos(W):
        return jnp.pad(W.astype(jnp.float32), ((0, 13), (0, 0)))
    Wzp, Wrp, Wqp = padpos(Wz_pos), padpos(Wr_pos), padpos(Wq_pos)
    bz2, br2, bq2 = (b.astype(jnp.float32).reshape(1, CIN)
                     for b in (bz_pos, br_pos, bq_pos))
    bzl2, brl2, bql2 = (b.astype(jnp.float32).reshape(1, H)
                        for b in (bz_lin, br_lin, bq_lin))

    grid = Npad // BN

    def zr_body(nxyz_ref, nhx_ref, xyz_ref, hT_ref,
                wzp_ref, bz_ref, wzl_ref, bzl_ref,
                wrp_ref, br_ref, wrl_ref, brl_ref,
                z_ref, rh_ref):
        nxyz = nxyz_ref[:, :16]
        rel = (nxyz.reshape(BN, k, 16) - xyz_ref[:, :16][:, None, :]).reshape(BN * k, 16)
        nhx = nhx_ref[...]
        wz = jnp.tanh(jnp.dot(rel, wzp_ref[...],
                              preferred_element_type=jnp.float32) + bz_ref[...])
        aggz = (wz * nhx).reshape(BN, k, CIN).sum(axis=1)
        zz = jax.nn.sigmoid(jnp.dot(aggz, wzl_ref[...],
                                    preferred_element_type=jnp.float32) + bzl_ref[...])
        wr = jnp.tanh(jnp.dot(rel, wrp_ref[...],
                              preferred_element_type=jnp.float32) + br_ref[...])
        aggr = (wr * nhx).reshape(BN, k, CIN).sum(axis=1)
        rr = jax.nn.sigmoid(jnp.dot(aggr, wrl_ref[...],
                                    preferred_element_type=jnp.float32) + brl_ref[...])
        z_ref[...] = zz
        rh = rr * hT_ref[...]
        rh_ref[...] = jnp.concatenate([rh, jnp.zeros_like(rh)], axis=1)

    full = lambda shape: pl.BlockSpec(shape, lambda i: (0, 0))
    z_nm, rh_nm = pl.pallas_call(
        zr_body,
        grid=(grid,),
        in_specs=[
            pl.BlockSpec((BN * k, 128), lambda i: (i, 0)),
            pl.BlockSpec((BN * k, CIN), lambda i: (i, 0)),
            pl.BlockSpec((BN, 128), lambda i: (i, 0)),
            pl.BlockSpec((BN, H), lambda i: (i, 0)),
            full((16, CIN)), full((1, CIN)), full((CIN, H)), full((1, H)),
            full((16, CIN)), full((1, CIN)), full((CIN, H)), full((1, H)),
        ],
        out_specs=[
            pl.BlockSpec((BN, H), lambda i: (i, 0)),
            pl.BlockSpec((BN, 2 * H), lambda i: (i, 0)),
        ],
        out_shape=[
            jax.ShapeDtypeStruct((Npad, H), jnp.float32),
            jax.ShapeDtypeStruct((Npad, 2 * H), jnp.float32),
        ],
    )(nbr_xyz, nbr_hx, xyzp, hTp,
      Wzp, bz2, Wz_lin.astype(jnp.float32), bzl2,
      Wrp, br2, Wr_lin.astype(jnp.float32), brl2)

    (nbr_rh,) = _sc_gather(rh_nm, idx_flat, 2 * H)

    def q_body(nxyz_ref, nrh_ref, nx_ref, xyz_ref, hT_ref, z_ref,
               wqp_ref, bq_ref, wql_ref, bql_ref, out_ref):
        nxyz = nxyz_ref[:, :16]
        rel = (nxyz.reshape(BN, k, 16) - xyz_ref[:, :16][:, None, :]).reshape(BN * k, 16)
        wq = jnp.tanh(jnp.dot(rel, wqp_ref[...],
                              preferred_element_type=jnp.float32) + bq_ref[...])
        nfeat = jnp.concatenate([nrh_ref[:, :H], nx_ref[:, H:]], axis=1)
        aggq = (wq * nfeat).reshape(BN, k, CIN).sum(axis=1)
        qq = jnp.tanh(jnp.dot(aggq, wql_ref[...],
                              preferred_element_type=jnp.float32) + bql_ref[...])
        zz = z_ref[...]
        out_ref[...] = (1.0 - zz) * hT_ref[...] + zz * qq

    h_new_nm = pl.pallas_call(
        q_body,
        grid=(grid,),
        in_specs=[
            pl.BlockSpec((BN * k, 128), lambda i: (i, 0)),
            pl.BlockSpec((BN * k, 2 * H), lambda i: (i, 0)),
            pl.BlockSpec((BN * k, CIN), lambda i: (i, 0)),
            pl.BlockSpec((BN, 128), lambda i: (i, 0)),
            pl.BlockSpec((BN, H), lambda i: (i, 0)),
            pl.BlockSpec((BN, H), lambda i: (i, 0)),
            full((16, CIN)), full((1, CIN)), full((CIN, H)), full((1, H)),
        ],
        out_specs=pl.BlockSpec((BN, H), lambda i: (i, 0)),
        out_shape=jax.ShapeDtypeStruct((Npad, H), jnp.float32),
    )(nbr_xyz, nbr_rh, nbr_hx, xyzp, hTp, z_nm,
      Wqp, bq2, Wq_lin.astype(jnp.float32), bql2)

    return h_new_nm[:N].T[None]


# speedup vs baseline: 7.9137x; 1.0904x over previous
import functools

import jax
import jax.numpy as jnp
from jax import lax
from jax.experimental import pallas as pl
from jax.experimental.pallas import tpu as pltpu
from jax.experimental.pallas import tpu_sc as plsc

NW = 32
CHUNK = 448
ITERS = 28
BN = 512
NKPAD = NW * CHUNK * ITERS


def _sc_gather(table, idx):
    per_w = NKPAD // NW
    nc = 2

    mesh = plsc.VectorSubcoreMesh(core_axis_name="c", subcore_axis_name="s")

    @functools.partial(
        pl.kernel, mesh=mesh,
        out_type=jax.ShapeDtypeStruct((NKPAD, 128), jnp.float32),
        scratch_types=[
            pltpu.VMEM((per_w,), jnp.int32),
            pltpu.VMEM((CHUNK, 128), jnp.float32),
            pltpu.VMEM((CHUNK, 128), jnp.float32),
            pltpu.SemaphoreType.DMA,
            pltpu.SemaphoreType.DMA,
        ],
    )
    def gather_k(idx_hbm, tab_hbm, out_hbm, idx_all, buf0, buf1, sem0, sem1):
        wid = lax.axis_index("s") * nc + lax.axis_index("c")
        base = wid * per_w
        pltpu.sync_copy(idx_hbm.at[pl.ds(base, per_w)], idx_all)

        bufs = (buf0, buf1)
        sems = (sem0, sem1)

        def gcopy(j, b):
            return pltpu.make_async_copy(
                tab_hbm.at[idx_all.at[pl.ds(j * CHUNK, CHUNK)]], bufs[b], sems[b])

        def step(j, b):
            @pl.when(j + 1 < ITERS)
            def _():
                gcopy(j + 1, 1 - b).start()
            gcopy(j, b).wait()
            pltpu.sync_copy(bufs[b], out_hbm.at[pl.ds(base + j * CHUNK, CHUNK)])

        gcopy(0, 0).start()

        def pair(i2, carry):
            step(i2 * 2, 0)
            step(i2 * 2 + 1, 1)
            return carry

        lax.fori_loop(0, ITERS // 2, pair, 0)

    return gather_k(idx, table)


def _unpack_bf16_pair(words):
    u = lax.bitcast_convert_type(words, jnp.uint32)
    hi = lax.bitcast_convert_type(u & jnp.uint32(0xFFFF0000), jnp.float32)
    lo = lax.bitcast_convert_type(u << 16, jnp.float32)
    return jnp.concatenate([hi, lo], axis=1)


def kernel(xyz, h, x, knn_indices,
           Wz_pos, bz_pos, Wz_lin, bz_lin,
           Wr_pos, br_pos, Wr_lin, br_lin,
           Wq_pos, bq_pos, Wq_lin, bq_lin):
    h = h.astype(jnp.float32)
    x = x.astype(jnp.float32)
    B, N, _ = xyz.shape
    k = knn_indices.shape[2]
    H = h.shape[1]
    CIN = 2 * H
    Npad = NKPAD // k
    grid = Npad // BN

    hT = h[0].T
    xT = x[0].T
    hx_bf = jnp.concatenate([hT, xT], axis=1).astype(jnp.bfloat16)
    u_hi = lax.bitcast_convert_type(hx_bf[:, :H], jnp.uint16).astype(jnp.uint32)
    u_lo = lax.bitcast_convert_type(hx_bf[:, H:], jnp.uint16).astype(jnp.uint32)
    packed = lax.bitcast_convert_type((u_hi << 16) | u_lo, jnp.float32)
    tab1 = jnp.pad(jnp.concatenate([packed, xyz[0]], axis=1),
                   ((0, Npad - N), (0, 128 - H - 3)))
    xyzc = jnp.pad(xyz[0], ((0, Npad - N), (0, 13)))
    hTp = jnp.pad(hT, ((0, Npad - N), (0, 0)))
    xTp = jnp.pad(xT, ((0, Npad - N), (0, 0)))
    idx_flat = jnp.pad(knn_indices[0].astype(jnp.int32).reshape(N * k),
                       (0, NKPAD - N * k))

    def padpos(W, rows):
        return jnp.pad(W.astype(jnp.float32), ((0, rows - 3), (0, 0)))
    bz2, br2, bq2 = (b.astype(jnp.float32).reshape(1, CIN)
                     for b in (bz_pos, br_pos, bq_pos))
    bzl2, brl2, bql2 = (b.astype(jnp.float32).reshape(1, H)
                        for b in (bz_lin, br_lin, bq_lin))

    nbr1 = _sc_gather(tab1, idx_flat)

    def zr_body(nbr_ref, xyzc_ref, hT_ref, xT_ref,
                wzp_ref, bz_ref, wzl_ref, bzl_ref,
                wrp_ref, br_ref, wrl_ref, brl_ref,
                z_ref, tab2_ref, rel_ref):
        nb = nbr_ref[...]
        feat = _unpack_bf16_pair(nb[:, :H])
        relf = (nb[:, H:H + 16].reshape(BN, k, 16)
                - xyzc_ref[...][:, None, :]).reshape(BN * k, 16)
        wz = jnp.tanh(jnp.dot(relf, wzp_ref[...],
                              preferred_element_type=jnp.float32) + bz_ref[...])
        aggz = (wz * feat).reshape(BN, k, CIN).sum(axis=1)
        zz = jax.nn.sigmoid(jnp.dot(aggz, wzl_ref[...],
                                    preferred_element_type=jnp.float32) + bzl_ref[...])
        wr = jnp.tanh(jnp.dot(relf, wrp_ref[...],
                              preferred_element_type=jnp.float32) + br_ref[...])
        aggr = (wr * feat).reshape(BN, k, CIN).sum(axis=1)
        rr = jax.nn.sigmoid(jnp.dot(aggr, wrl_ref[...],
                                    preferred_element_type=jnp.float32) + brl_ref[...])
        z_ref[...] = zz
        tab2_ref[...] = jnp.concatenate([rr * hT_ref[...], xT_ref[...]], axis=1)
        rel_ref[...] = relf[:, :8]

    full = lambda shape: pl.BlockSpec(shape, lambda i: (0, 0))
    z_nm, tab2, rel8 = pl.pallas_call(
        zr_body,
        grid=(grid,),
        in_specs=[
            pl.BlockSpec((BN * k, 128), lambda i: (i, 0)),
            pl.BlockSpec((BN, 16), lambda i: (i, 0)),
            pl.BlockSpec((BN, H), lambda i: (i, 0)),
            pl.BlockSpec((BN, H), lambda i: (i, 0)),
            full((16, CIN)), full((1, CIN)), full((CIN, H)), full((1, H)),
            full((16, CIN)), full((1, CIN)), full((CIN, H)), full((1, H)),
        ],
        out_specs=[
            pl.BlockSpec((BN, H), lambda i: (i, 0)),
            pl.BlockSpec((BN, CIN), lambda i: (i, 0)),
            pl.BlockSpec((BN * k, 8), lambda i: (i, 0)),
        ],
        out_shape=[
            jax.ShapeDtypeStruct((Npad, H), jnp.float32),
            jax.ShapeDtypeStruct((Npad, CIN), jnp.float32),
            jax.ShapeDtypeStruct((NKPAD, 8), jnp.float32),
        ],
    )(nbr1, xyzc, hTp, xTp,
      padpos(Wz_pos, 16), bz2, Wz_lin.astype(jnp.float32), bzl2,
      padpos(Wr_pos, 16), br2, Wr_lin.astype(jnp.float32), brl2)

    nbr2 = _sc_gather(tab2, idx_flat)

    def q_body(nbr_ref, rel_ref, z_ref, hT_ref,
               wqp_ref, bq_ref, wql_ref, bql_ref, out_ref):
        wq = jnp.tanh(jnp.dot(rel_ref[...], wqp_ref[...],
                              preferred_element_type=jnp.float32) + bq_ref[...])
        aggq = (wq * nbr_ref[...]).reshape(BN, k, CIN).sum(axis=1)
        qq = jnp.tanh(jnp.dot(aggq, wql_ref[...],
                              preferred_element_type=jnp.float32) + bql_ref[...])
        zz = z_ref[...]
        out_ref[...] = (1.0 - zz) * hT_ref[...] + zz * qq

    h_new_nm = pl.pallas_call(
        q_body,
        grid=(grid,),
        in_specs=[
            pl.BlockSpec((BN * k, CIN), lambda i: (i, 0)),
            pl.BlockSpec((BN * k, 8), lambda i: (i, 0)),
            pl.BlockSpec((BN, H), lambda i: (i, 0)),
            pl.BlockSpec((BN, H), lambda i: (i, 0)),
            full((8, CIN)), full((1, CIN)), full((CIN, H)), full((1, H)),
        ],
        out_specs=pl.BlockSpec((BN, H), lambda i: (i, 0)),
        out_shape=jax.ShapeDtypeStruct((Npad, H), jnp.float32),
    )(nbr2, rel8, z_nm, hTp,
      padpos(Wq_pos, 8), bq2, Wq_lin.astype(jnp.float32), bql2)

    return h_new_nm[:N].T[None]


# speedup vs baseline: 9.5183x; 1.2028x over previous
import functools

import jax
import jax.numpy as jnp
from jax import lax
from jax.experimental import pallas as pl
from jax.experimental.pallas import tpu as pltpu
from jax.experimental.pallas import tpu_sc as plsc

NW = 32
CHUNK = 448
ITERS = 28
BN = 512
NKPAD = NW * CHUNK * ITERS


def _sc_gather(table, idx):
    per_w = NKPAD // NW
    nc = 2

    mesh = plsc.VectorSubcoreMesh(core_axis_name="c", subcore_axis_name="s")

    @functools.partial(
        pl.kernel, mesh=mesh,
        out_type=jax.ShapeDtypeStruct((NKPAD, 128), jnp.float32),
        scratch_types=[
            pltpu.VMEM((per_w,), jnp.int32),
            pltpu.VMEM((CHUNK, 128), jnp.float32),
            pltpu.VMEM((CHUNK, 128), jnp.float32),
            pltpu.SemaphoreType.DMA,
            pltpu.SemaphoreType.DMA,
        ],
    )
    def gather_k(idx_hbm, tab_hbm, out_hbm, idx_all, buf0, buf1, sem0, sem1):
        wid = lax.axis_index("s") * nc + lax.axis_index("c")
        base = wid * per_w
        pltpu.sync_copy(idx_hbm.at[pl.ds(base, per_w)], idx_all)

        bufs = (buf0, buf1)
        sems = (sem0, sem1)

        def gcopy(j, b):
            return pltpu.make_async_copy(
                tab_hbm.at[idx_all.at[pl.ds(j * CHUNK, CHUNK)]], bufs[b], sems[b])

        def step(j, b):
            @pl.when(j + 1 < ITERS)
            def _():
                gcopy(j + 1, 1 - b).start()
            gcopy(j, b).wait()
            pltpu.sync_copy(bufs[b], out_hbm.at[pl.ds(base + j * CHUNK, CHUNK)])

        gcopy(0, 0).start()

        def pair(i2, carry):
            step(i2 * 2, 0)
            step(i2 * 2 + 1, 1)
            return carry

        lax.fori_loop(0, ITERS // 2, pair, 0)

    return gather_k(idx, table)


def _unpack_bf16_pair(words):
    u = lax.bitcast_convert_type(words, jnp.uint32)
    hi = lax.bitcast_convert_type(u & jnp.uint32(0xFFFF0000), jnp.float32)
    lo = lax.bitcast_convert_type(u << 16, jnp.float32)
    return jnp.concatenate([hi, lo], axis=1)


def kernel(xyz, h, x, knn_indices,
           Wz_pos, bz_pos, Wz_lin, bz_lin,
           Wr_pos, br_pos, Wr_lin, br_lin,
           Wq_pos, bq_pos, Wq_lin, bq_lin):
    h = h.astype(jnp.float32)
    x = x.astype(jnp.float32)
    B, N, _ = xyz.shape
    k = knn_indices.shape[2]
    H = h.shape[1]
    CIN = 2 * H
    Npad = NKPAD // k
    grid = Npad // BN

    hT = h[0].T
    xT = x[0].T
    hx_bf = jnp.concatenate([hT, xT], axis=1).astype(jnp.bfloat16)
    u_hi = lax.bitcast_convert_type(hx_bf[:, :H], jnp.uint16).astype(jnp.uint32)
    u_lo = lax.bitcast_convert_type(hx_bf[:, H:], jnp.uint16).astype(jnp.uint32)
    packed = lax.bitcast_convert_type((u_hi << 16) | u_lo, jnp.float32)
    tab1 = jnp.pad(jnp.concatenate([packed, xyz[0]], axis=1),
                   ((0, Npad - N), (0, 128 - H - 3)))
    xyzc = jnp.pad(xyz[0], ((0, Npad - N), (0, 13)))
    hTp = jnp.pad(hT, ((0, Npad - N), (0, 0)))
    xTp = jnp.pad(xT, ((0, Npad - N), (0, 0)))
    idx_flat = jnp.pad(knn_indices[0].astype(jnp.int32).T,
                       ((0, 0), (0, Npad - N))).reshape(NKPAD)

    def padpos(W, rows):
        return jnp.pad(W.astype(jnp.float32), ((0, rows - 3), (0, 0)))
    bz2, br2, bq2 = (b.astype(jnp.float32).reshape(1, CIN)
                     for b in (bz_pos, br_pos, bq_pos))
    bzl2, brl2, bql2 = (b.astype(jnp.float32).reshape(1, H)
                        for b in (bz_lin, br_lin, bq_lin))

    nbr1 = _sc_gather(tab1, idx_flat).reshape(k, Npad, 128)

    def zr_body(nbr_ref, xyzc_ref, hT_ref, xT_ref,
                wzp_ref, bz_ref, wzl_ref, bzl_ref,
                wrp_ref, br_ref, wrl_ref, brl_ref,
                z_ref, tab2_ref, rel_ref):
        nb = nbr_ref[...]
        feats = [_unpack_bf16_pair(nb[j, :, :H]) for j in range(k)]
        xc = xyzc_ref[...]
        relf = jnp.concatenate([nb[j, :, H:H + 16] - xc for j in range(k)],
                               axis=0)
        wz = jnp.tanh(jnp.dot(relf, wzp_ref[...],
                              preferred_element_type=jnp.float32) + bz_ref[...])
        aggz = sum(wz[j * BN:(j + 1) * BN] * feats[j] for j in range(k))
        zz = jax.nn.sigmoid(jnp.dot(aggz, wzl_ref[...],
                                    preferred_element_type=jnp.float32) + bzl_ref[...])
        wr = jnp.tanh(jnp.dot(relf, wrp_ref[...],
                              preferred_element_type=jnp.float32) + br_ref[...])
        aggr = sum(wr[j * BN:(j + 1) * BN] * feats[j] for j in range(k))
        rr = jax.nn.sigmoid(jnp.dot(aggr, wrl_ref[...],
                                    preferred_element_type=jnp.float32) + brl_ref[...])
        z_ref[...] = zz
        tab2_ref[...] = jnp.concatenate([rr * hT_ref[...], xT_ref[...]], axis=1)
        rel_ref[...] = relf[:, :8].reshape(k, BN, 8)

    full = lambda shape: pl.BlockSpec(shape, lambda i: (0, 0))
    z_nm, tab2, rel8 = pl.pallas_call(
        zr_body,
        grid=(grid,),
        in_specs=[
            pl.BlockSpec((k, BN, 128), lambda i: (0, i, 0)),
            pl.BlockSpec((BN, 16), lambda i: (i, 0)),
            pl.BlockSpec((BN, H), lambda i: (i, 0)),
            pl.BlockSpec((BN, H), lambda i: (i, 0)),
            full((16, CIN)), full((1, CIN)), full((CIN, H)), full((1, H)),
            full((16, CIN)), full((1, CIN)), full((CIN, H)), full((1, H)),
        ],
        out_specs=[
            pl.BlockSpec((BN, H), lambda i: (i, 0)),
            pl.BlockSpec((BN, CIN), lambda i: (i, 0)),
            pl.BlockSpec((k, BN, 8), lambda i: (0, i, 0)),
        ],
        out_shape=[
            jax.ShapeDtypeStruct((Npad, H), jnp.float32),
            jax.ShapeDtypeStruct((Npad, CIN), jnp.float32),
            jax.ShapeDtypeStruct((k, Npad, 8), jnp.float32),
        ],
    )(nbr1, xyzc, hTp, xTp,
      padpos(Wz_pos, 16), bz2, Wz_lin.astype(jnp.float32), bzl2,
      padpos(Wr_pos, 16), br2, Wr_lin.astype(jnp.float32), brl2)

    nbr2 = _sc_gather(tab2, idx_flat).reshape(k, Npad, CIN)

    def q_body(nbr_ref, rel_ref, z_ref, hT_ref,
               wqp_ref, bq_ref, wql_ref, bql_ref, out_ref):
        nb = nbr_ref[...]
        wq = jnp.tanh(jnp.dot(rel_ref[...].reshape(k * BN, 8), wqp_ref[...],
                              preferred_element_type=jnp.float32) + bq_ref[...])
        aggq = sum(wq[j * BN:(j + 1) * BN] * nb[j] for j in range(k))
        qq = jnp.tanh(jnp.dot(aggq, wql_ref[...],
                              preferred_element_type=jnp.float32) + bql_ref[...])
        zz = z_ref[...]
        out_ref[...] = (1.0 - zz) * hT_ref[...] + zz * qq

    h_new_nm = pl.pallas_call(
        q_body,
        grid=(grid,),
        in_specs=[
            pl.BlockSpec((k, BN, CIN), lambda i: (0, i, 0)),
            pl.BlockSpec((k, BN, 8), lambda i: (0, i, 0)),
            pl.BlockSpec((BN, H), lambda i: (i, 0)),
            pl.BlockSpec((BN, H), lambda i: (i, 0)),
            full((8, CIN)), full((1, CIN)), full((CIN, H)), full((1, H)),
        ],
        out_specs=pl.BlockSpec((BN, H), lambda i: (i, 0)),
        out_shape=jax.ShapeDtypeStruct((Npad, H), jnp.float32),
    )(nbr2, rel8, z_nm, hTp,
      padpos(Wq_pos, 8), bq2, Wq_lin.astype(jnp.float32), bql2)

    return h_new_nm[:N].T[None]
